# baseline (speedup 1.0000x reference)
import functools

import jax
import jax.numpy as jnp
from jax import lax
from jax.experimental import pallas as pl
from jax.experimental.pallas import tpu as pltpu
from jax.experimental.pallas import tpu_sc as plsc

_BETA = 0.5
_N = 4096
_B = 2048
_NE = 2 * _B
_NC = 2
_NS = 16
_NW = _NC * _NS
_EPW = _NE // _NW
_GB = 2
_NB = _EPW // _GB
_LANES = 16
_CH = _N // _LANES
_UNROLL = 4
_GRP = 8
_WPACK = 512


def _bf16_round(x):
    u = lax.bitcast_convert_type(x, jnp.uint32)
    u = u + jnp.uint32(0x7FFF) + ((u >> jnp.uint32(16)) & jnp.uint32(1))
    return lax.bitcast_convert_type(u & jnp.uint32(0xFFFF0000), jnp.float32)


def _rsqrt(x):
    u = lax.bitcast_convert_type(x, jnp.uint32)
    y = lax.bitcast_convert_type(jnp.uint32(0x5F3759DF) - (u >> jnp.uint32(1)),
                                 jnp.float32)
    xh = x * 0.5
    for _ in range(3):
        y = y * (1.5 - xh * y * y)
    return y


def _sc_maa(A0, A1, idx, w0, w1, wts):
    mesh = plsc.VectorSubcoreMesh(core_axis_name="c", subcore_axis_name="s")

    @functools.partial(
        pl.kernel,
        out_type=jax.ShapeDtypeStruct((_NE,), jnp.float32),
        mesh=mesh,
        scratch_types=[
            pltpu.VMEM((_GB * 2, _N), jnp.float32),
            pltpu.VMEM((_GB * 2, _N), jnp.float32),
            pltpu.VMEM((_GB * 2, _N), jnp.float32),
            pltpu.VMEM((_GB * 2, _N), jnp.float32),
            pltpu.VMEM((_N,), jnp.float32),
            pltpu.VMEM((_N,), jnp.float32),
            pltpu.VMEM((_NB, 2 * _GB), jnp.int32),
            pltpu.VMEM((_WPACK,), jnp.float32),
            pltpu.VMEM((_EPW,), jnp.float32),
            pltpu.SemaphoreType.DMA,
            pltpu.SemaphoreType.DMA,
        ],
    )
    def body(A0h, A1h, idxh, w0h, w1h, wth, out_h,
             a0b0, a0b1, a1b0, a1b1, wv0, wv1, idx_v, wt_v, res_v,
             sem0, sem1):
        wid = lax.axis_index("s") * _NC + lax.axis_index("c")
        bufs = ((a0b0, a1b0, sem0), (a0b1, a1b1, sem1))

        pltpu.sync_copy(idxh.at[wid], idx_v)
        pltpu.sync_copy(w0h, wv0)
        pltpu.sync_copy(w1h, wv1)
        pltpu.sync_copy(wth.at[wid // _NS], wt_v)
        for base in (128, 336):
            for k in range(4):
                sl = pl.ds(base + 16 * k, _LANES)
                wt_v[sl] = _bf16_round(wt_v[sl])

        def gathers(p, b):
            b0, b1, sem = bufs[p]
            return (pltpu.make_async_copy(A0h.at[idx_v.at[b]], b0, sem),
                    pltpu.make_async_copy(A1h.at[idx_v.at[b]], b1, sem))

        def start(p, b):
            g0, g1 = gathers(p, b)
            g0.start()
            g1.start()

        def wait(p, b):
            g0, g1 = gathers(p, b)
            g0.wait()
            g1.wait()

        start(0, 0)
        start(1, 1)

        zeros = jnp.zeros((_LANES,), jnp.float32)
        lanes = lax.iota(jnp.int32, _LANES)

        def allsum(x):
            for s in (8, 4, 2, 1):
                x = x + x.at[jnp.bitwise_xor(lanes, s)].get(
                    mode="promise_in_bounds")
            return x

        def batch_accs(b0, b1):
            def chunk(k, accs):
                accs = list(accs)
                for u in range(_UNROLL):
                    csl = pl.ds((k * _UNROLL + u) * _LANES, _LANES)
                    w0c = wv0[csl]
                    w1c = wv1[csl]
                    for i in range(_GB):
                        o = 6 * i
                        b0s = b0[2 * i, csl] * w0c
                        b1s = b1[2 * i, csl] * w1c
                        b0d = b0[2 * i + 1, csl] * w0c
                        b1d = b1[2 * i + 1, csl] * w1c
                        accs[o] = accs[o] + b0s * b1d
                        accs[o + 1] = accs[o + 1] + b1s * b0d
                        accs[o + 2] = accs[o + 2] + b0s * b0s
                        accs[o + 3] = accs[o + 3] + b1s * b1s
                        accs[o + 4] = accs[o + 4] + b0d * b0d
                        accs[o + 5] = accs[o + 5] + b1d * b1d
                return tuple(accs)
            return lax.fori_loop(0, _CH // _UNROLL, chunk, (zeros,) * 12)

        def mlp(x, base):
            acc = zeros
            for k in range(4):
                h = jnp.maximum(x * wt_v[pl.ds(base + 16 * k, _LANES)]
                                + wt_v[pl.ds(base + 64 + 16 * k, _LANES)],
                                0.0)
                acc = acc + _bf16_round(h) * wt_v[pl.ds(base + 128 + 16 * k,
                                                        _LANES)]
            return allsum(acc) + wt_v[pl.ds(base + 192, _LANES)]

        def finish_edge(accs):
            f, r, s0s, s1s, s0d, s1d = [allsum(a) for a in accs]
            on = (f * _rsqrt(s0s * s1d) + r * _rsqrt(s1s * s0d))
            sl = _BETA * mlp(on, 0)
            return mlp(sl, 208)

        def run_batch(b, j, vec):
            p = j % 2
            wait(p, b)
            b0, b1, _ = bufs[p][:3]
            accs = batch_accs(b0, b1)
            for i in range(_GB):
                o = finish_edge(accs[6 * i:6 * i + 6])
                vec = jnp.where(lanes == (_GB * j + i), o, vec)
            nb = b + 2

            @pl.when(nb < _NB)
            def _():
                start(p, nb)
            return vec

        def g_body(g, carry):
            vec = zeros
            for j in range(_GRP):
                vec = run_batch(_GRP * g + j, j, vec)
            res_v[pl.ds(g * _GRP * _GB, _GRP * _GB)] = vec
            return carry

        lax.fori_loop(0, _NB // _GRP, g_body, 0)
        pltpu.sync_copy(res_v, out_h.at[pl.ds(wid * _EPW, _EPW)])

    return body(A0, A1, idx, w0, w1, wts)


def _pack_layer(w1, b1, w2, b2, v1, c1, v2, c2):
    parts = [w1.reshape(64), b1.reshape(64), w2.reshape(64),
             jnp.broadcast_to(b2.reshape(1), (_LANES,)),
             v1.reshape(64), c1.reshape(64), v2.reshape(64),
             jnp.broadcast_to(c2.reshape(1), (_LANES,)),
             jnp.zeros((_WPACK - 416,), jnp.float32)]
    return jnp.concatenate(parts)


def kernel(A0, A1, feats0, feats1, out0, out1, src0, dst0, src1, dst1,
           p1w1_0, p1b1_0, p1w2_0, p1b2_0, p2w1_0, p2b1_0, p2w2_0, p2b2_0,
           p1w1_1, p1b1_1, p1w2_1, p1b2_1, p2w1_1, p2b1_1, p2w2_1, p2b2_1):
    src = jnp.concatenate([src0, src1])
    dst = jnp.concatenate([dst0, dst1])
    idx = jnp.stack([src, dst], axis=1)
    idx = idx.reshape(_NW, _NB, 2 * _GB).astype(jnp.int32)
    w0 = feats0.reshape(_N)
    w1 = feats1.reshape(_N)
    wts = jnp.stack([
        _pack_layer(p1w1_0, p1b1_0, p1w2_0, p1b2_0,
                    p2w1_0, p2b1_0, p2w2_0, p2b2_0),
        _pack_layer(p1w1_1, p1b1_1, p1w2_1, p1b2_1,
                    p2w1_1, p2b1_1, p2w2_1, p2b2_1),
    ])
    out = _sc_maa(A0, A1, idx, w0, w1, wts)
    return out.reshape(2, _B, 1)


# speedup vs baseline: 1.0380x; 1.0380x over previous
import functools

import jax
import jax.numpy as jnp
from jax import lax
from jax.experimental import pallas as pl
from jax.experimental.pallas import tpu as pltpu
from jax.experimental.pallas import tpu_sc as plsc

_BETA = 0.5
_N = 4096
_B = 2048
_NE = 2 * _B
_NC = 2
_NS = 16
_NW = _NC * _NS
_EPW = _NE // _NW
_GB = 2
_NB = _EPW // _GB
_LANES = 16
_CH = _N // _LANES
_UNROLL = 8
_GRP = 8
_WPACK = 512


def _bf16_round(x):
    u = lax.bitcast_convert_type(x, jnp.uint32)
    u = u + jnp.uint32(0x7FFF) + ((u >> jnp.uint32(16)) & jnp.uint32(1))
    return lax.bitcast_convert_type(u & jnp.uint32(0xFFFF0000), jnp.float32)


def _rsqrt(x):
    u = lax.bitcast_convert_type(x, jnp.uint32)
    y = lax.bitcast_convert_type(jnp.uint32(0x5F3759DF) - (u >> jnp.uint32(1)),
                                 jnp.float32)
    xh = x * 0.5
    for _ in range(3):
        y = y * (1.5 - xh * y * y)
    return y


def _sc_maa(A0, A1, idx, w0, w1, wts):
    mesh = plsc.VectorSubcoreMesh(core_axis_name="c", subcore_axis_name="s")

    @functools.partial(
        pl.kernel,
        out_type=jax.ShapeDtypeStruct((_NE,), jnp.float32),
        mesh=mesh,
        scratch_types=[
            pltpu.VMEM((_GB * 2, _N), jnp.float32),
            pltpu.VMEM((_GB * 2, _N), jnp.float32),
            pltpu.VMEM((_GB * 2, _N), jnp.float32),
            pltpu.VMEM((_GB * 2, _N), jnp.float32),
            pltpu.VMEM((_N,), jnp.float32),
            pltpu.VMEM((_N,), jnp.float32),
            pltpu.VMEM((_NB, 2 * _GB), jnp.int32),
            pltpu.VMEM((_WPACK,), jnp.float32),
            pltpu.VMEM((_EPW,), jnp.float32),
            pltpu.SemaphoreType.DMA,
            pltpu.SemaphoreType.DMA,
        ],
    )
    def body(A0h, A1h, idxh, w0h, w1h, wth, out_h,
             a0b0, a0b1, a1b0, a1b1, wv0, wv1, idx_v, wt_v, res_v,
             sem0, sem1):
        wid = lax.axis_index("s") * _NC + lax.axis_index("c")
        bufs = ((a0b0, a1b0, sem0), (a0b1, a1b1, sem1))

        pltpu.sync_copy(idxh.at[wid], idx_v)
        pltpu.sync_copy(w0h, wv0)
        pltpu.sync_copy(w1h, wv1)
        pltpu.sync_copy(wth.at[wid // _NS], wt_v)
        for base in (128, 336):
            for k in range(4):
                sl = pl.ds(base + 16 * k, _LANES)
                wt_v[sl] = _bf16_round(wt_v[sl])

        def gathers(p, b):
            b0, b1, sem = bufs[p]
            return (pltpu.make_async_copy(A0h.at[idx_v.at[b]], b0, sem),
                    pltpu.make_async_copy(A1h.at[idx_v.at[b]], b1, sem))

        def start(p, b):
            g0, g1 = gathers(p, b)
            g0.start()
            g1.start()

        def wait(p, b):
            g0, g1 = gathers(p, b)
            g0.wait()
            g1.wait()

        start(0, 0)
        start(1, 1)

        zeros = jnp.zeros((_LANES,), jnp.float32)
        lanes = lax.iota(jnp.int32, _LANES)

        def allsum(x):
            for s in (8, 4, 2, 1):
                x = x + x.at[jnp.bitwise_xor(lanes, s)].get(
                    mode="promise_in_bounds")
            return x

        def edge_accs(b0, b1, i):
            def chunk(k, accs):
                f, r, s0s, s1s, s0d, s1d = accs
                for u in range(_UNROLL):
                    csl = pl.ds((k * _UNROLL + u) * _LANES, _LANES)
                    w0c = wv0[csl]
                    w1c = wv1[csl]
                    b0s = b0[2 * i, csl] * w0c
                    b1s = b1[2 * i, csl] * w1c
                    b0d = b0[2 * i + 1, csl] * w0c
                    b1d = b1[2 * i + 1, csl] * w1c
                    f = f + b0s * b1d
                    r = r + b1s * b0d
                    s0s = s0s + b0s * b0s
                    s1s = s1s + b1s * b1s
                    s0d = s0d + b0d * b0d
                    s1d = s1d + b1d * b1d
                return (f, r, s0s, s1s, s0d, s1d)
            return lax.fori_loop(0, _CH // _UNROLL, chunk, (zeros,) * 6)

        def mlp(x, base):
            acc = zeros
            for k in range(4):
                h = jnp.maximum(x * wt_v[pl.ds(base + 16 * k, _LANES)]
                                + wt_v[pl.ds(base + 64 + 16 * k, _LANES)],
                                0.0)
                acc = acc + _bf16_round(h) * wt_v[pl.ds(base + 128 + 16 * k,
                                                        _LANES)]
            return allsum(acc) + wt_v[pl.ds(base + 192, _LANES)]

        def finish_edge(accs):
            f, r, s0s, s1s, s0d, s1d = [allsum(a) for a in accs]
            on = (f * _rsqrt(s0s * s1d) + r * _rsqrt(s1s * s0d))
            sl = _BETA * mlp(on, 0)
            return mlp(sl, 208)

        def run_batch(b, j, vec):
            p = j % 2
            wait(p, b)
            b0, b1, _ = bufs[p][:3]
            for i in range(_GB):
                o = finish_edge(edge_accs(b0, b1, i))
                vec = jnp.where(lanes == (_GB * j + i), o, vec)
            nb = b + 2

            @pl.when(nb < _NB)
            def _():
                start(p, nb)
            return vec

        def g_body(g, carry):
            vec = zeros
            for j in range(_GRP):
                vec = run_batch(_GRP * g + j, j, vec)
            res_v[pl.ds(g * _GRP * _GB, _GRP * _GB)] = vec
            return carry

        lax.fori_loop(0, _NB // _GRP, g_body, 0)
        pltpu.sync_copy(res_v, out_h.at[pl.ds(wid * _EPW, _EPW)])

    return body(A0, A1, idx, w0, w1, wts)


def _pack_layer(w1, b1, w2, b2, v1, c1, v2, c2):
    parts = [w1.reshape(64), b1.reshape(64), w2.reshape(64),
             jnp.broadcast_to(b2.reshape(1), (_LANES,)),
             v1.reshape(64), c1.reshape(64), v2.reshape(64),
             jnp.broadcast_to(c2.reshape(1), (_LANES,)),
             jnp.zeros((_WPACK - 416,), jnp.float32)]
    return jnp.concatenate(parts)


def kernel(A0, A1, feats0, feats1, out0, out1, src0, dst0, src1, dst1,
           p1w1_0, p1b1_0, p1w2_0, p1b2_0, p2w1_0, p2b1_0, p2w2_0, p2b2_0,
           p1w1_1, p1b1_1, p1w2_1, p1b2_1, p2w1_1, p2b1_1, p2w2_1, p2b2_1):
    src = jnp.concatenate([src0, src1])
    dst = jnp.concatenate([dst0, dst1])
    idx = jnp.stack([src, dst], axis=1)
    idx = idx.reshape(_NW, _NB, 2 * _GB).astype(jnp.int32)
    w0 = feats0.reshape(_N)
    w1 = feats1.reshape(_N)
    wts = jnp.stack([
        _pack_layer(p1w1_0, p1b1_0, p1w2_0, p1b2_0,
                    p2w1_0, p2b1_0, p2w2_0, p2b2_0),
        _pack_layer(p1w1_1, p1b1_1, p1w2_1, p1b2_1,
                    p2w1_1, p2b1_1, p2w2_1, p2b2_1),
    ])
    out = _sc_maa(A0, A1, idx, w0, w1, wts)
    return out.reshape(2, _B, 1)


# speedup vs baseline: 1.0445x; 1.0062x over previous
import functools

import jax
import jax.numpy as jnp
from jax import lax
from jax.experimental import pallas as pl
from jax.experimental.pallas import tpu as pltpu
from jax.experimental.pallas import tpu_sc as plsc

_BETA = 0.5
_N = 4096
_B = 2048
_NE = 2 * _B
_NC = 2
_NS = 16
_NW = _NC * _NS
_EPW = _NE // _NW
_GB = 2
_NB = _EPW // _GB
_LANES = 16
_CH = _N // _LANES
_UNROLL = 4
_GRP = 8
_WPACK = 512


def _bf16_round(x):
    u = lax.bitcast_convert_type(x, jnp.uint32)
    u = u + jnp.uint32(0x7FFF) + ((u >> jnp.uint32(16)) & jnp.uint32(1))
    return lax.bitcast_convert_type(u & jnp.uint32(0xFFFF0000), jnp.float32)


def _rsqrt(x):
    u = lax.bitcast_convert_type(x, jnp.uint32)
    y = lax.bitcast_convert_type(jnp.uint32(0x5F3759DF) - (u >> jnp.uint32(1)),
                                 jnp.float32)
    xh = x * 0.5
    for _ in range(3):
        y = y * (1.5 - xh * y * y)
    return y


def _sc_maa(A0, A1, idx, w0, w1, wts):
    mesh = plsc.VectorSubcoreMesh(core_axis_name="c", subcore_axis_name="s")

    @functools.partial(
        pl.kernel,
        out_type=jax.ShapeDtypeStruct((_NE,), jnp.float32),
        mesh=mesh,
        scratch_types=[
            pltpu.VMEM((_GB * 2, _N), jnp.float32),
            pltpu.VMEM((_GB * 2, _N), jnp.float32),
            pltpu.VMEM((_GB * 2, _N), jnp.float32),
            pltpu.VMEM((_GB * 2, _N), jnp.float32),
            pltpu.VMEM((_N,), jnp.float32),
            pltpu.VMEM((_N,), jnp.float32),
            pltpu.VMEM((_NB, 2 * _GB), jnp.int32),
            pltpu.VMEM((_WPACK,), jnp.float32),
            pltpu.VMEM((_EPW,), jnp.float32),
            pltpu.SemaphoreType.DMA,
            pltpu.SemaphoreType.DMA,
        ],
    )
    def body(A0h, A1h, idxh, w0h, w1h, wth, out_h,
             a0b0, a0b1, a1b0, a1b1, wv0, wv1, idx_v, wt_v, res_v,
             sem0, sem1):
        wid = lax.axis_index("s") * _NC + lax.axis_index("c")
        bufs = ((a0b0, a1b0, sem0), (a0b1, a1b1, sem1))

        pltpu.sync_copy(idxh.at[wid], idx_v)
        pltpu.sync_copy(w0h, wv0)
        pltpu.sync_copy(w1h, wv1)
        pltpu.sync_copy(wth.at[wid // _NS], wt_v)
        for base in (128, 336):
            for k in range(4):
                sl = pl.ds(base + 16 * k, _LANES)
                wt_v[sl] = _bf16_round(wt_v[sl])

        def gathers(p, b):
            b0, b1, sem = bufs[p]
            return (pltpu.make_async_copy(A0h.at[idx_v.at[b]], b0, sem),
                    pltpu.make_async_copy(A1h.at[idx_v.at[b]], b1, sem))

        def start(p, b):
            g0, g1 = gathers(p, b)
            g0.start()
            g1.start()

        def wait(p, b):
            g0, g1 = gathers(p, b)
            g0.wait()
            g1.wait()

        start(0, 0)
        start(1, 1)

        zeros = jnp.zeros((_LANES,), jnp.float32)
        lanes = lax.iota(jnp.int32, _LANES)

        def allsum(x):
            for s in (8, 4, 2, 1):
                x = x + x.at[jnp.bitwise_xor(lanes, s)].get(
                    mode="promise_in_bounds")
            return x

        def edge_accs(b0, b1, i):
            def chunk(k, accs):
                f, r, s0s, s1s, s0d, s1d = accs
                for u in range(_UNROLL):
                    csl = pl.ds((k * _UNROLL + u) * _LANES, _LANES)
                    w0c = wv0[csl]
                    w1c = wv1[csl]
                    b0s = b0[2 * i, csl] * w0c
                    b1s = b1[2 * i, csl] * w1c
                    b0d = b0[2 * i + 1, csl] * w0c
                    b1d = b1[2 * i + 1, csl] * w1c
                    f = f + b0s * b1d
                    r = r + b1s * b0d
                    s0s = s0s + b0s * b0s
                    s1s = s1s + b1s * b1s
                    s0d = s0d + b0d * b0d
                    s1d = s1d + b1d * b1d
                return (f, r, s0s, s1s, s0d, s1d)
            return lax.fori_loop(0, _CH // _UNROLL, chunk, (zeros,) * 6)

        def mlp(x, base):
            acc = zeros
            for k in range(4):
                h = jnp.maximum(x * wt_v[pl.ds(base + 16 * k, _LANES)]
                                + wt_v[pl.ds(base + 64 + 16 * k, _LANES)],
                                0.0)
                acc = acc + _bf16_round(h) * wt_v[pl.ds(base + 128 + 16 * k,
                                                        _LANES)]
            return allsum(acc) + wt_v[pl.ds(base + 192, _LANES)]

        def finish_edge(accs):
            f, r, s0s, s1s, s0d, s1d = [allsum(a) for a in accs]
            on = (f * _rsqrt(s0s * s1d) + r * _rsqrt(s1s * s0d))
            sl = _BETA * mlp(on, 0)
            return mlp(sl, 208)

        def run_batch(b, j, vec):
            p = j % 2
            wait(p, b)
            b0, b1, _ = bufs[p][:3]
            for i in range(_GB):
                o = finish_edge(edge_accs(b0, b1, i))
                vec = jnp.where(lanes == (_GB * j + i), o, vec)
            nb = b + 2

            @pl.when(nb < _NB)
            def _():
                start(p, nb)
            return vec

        def g_body(g, carry):
            vec = zeros
            for j in range(_GRP):
                vec = run_batch(_GRP * g + j, j, vec)
            res_v[pl.ds(g * _GRP * _GB, _GRP * _GB)] = vec
            return carry

        lax.fori_loop(0, _NB // _GRP, g_body, 0)
        pltpu.sync_copy(res_v, out_h.at[pl.ds(wid * _EPW, _EPW)])

    return body(A0, A1, idx, w0, w1, wts)


def _pack_layer(w1, b1, w2, b2, v1, c1, v2, c2):
    parts = [w1.reshape(64), b1.reshape(64), w2.reshape(64),
             jnp.broadcast_to(b2.reshape(1), (_LANES,)),
             v1.reshape(64), c1.reshape(64), v2.reshape(64),
             jnp.broadcast_to(c2.reshape(1), (_LANES,)),
             jnp.zeros((_WPACK - 416,), jnp.float32)]
    return jnp.concatenate(parts)


def kernel(A0, A1, feats0, feats1, out0, out1, src0, dst0, src1, dst1,
           p1w1_0, p1b1_0, p1w2_0, p1b2_0, p2w1_0, p2b1_0, p2w2_0, p2b2_0,
           p1w1_1, p1b1_1, p1w2_1, p1b2_1, p2w1_1, p2b1_1, p2w2_1, p2b2_1):
    src = jnp.concatenate([src0, src1])
    dst = jnp.concatenate([dst0, dst1])
    idx = jnp.stack([src, dst], axis=1)
    idx = idx.reshape(_NW, _NB, 2 * _GB).astype(jnp.int32)
    w0 = feats0.reshape(_N)
    w1 = feats1.reshape(_N)
    wts = jnp.stack([
        _pack_layer(p1w1_0, p1b1_0, p1w2_0, p1b2_0,
                    p2w1_0, p2b1_0, p2w2_0, p2b2_0),
        _pack_layer(p1w1_1, p1b1_1, p1w2_1, p1b2_1,
                    p2w1_1, p2b1_1, p2w2_1, p2b2_1),
    ])
    out = _sc_maa(A0, A1, idx, w0, w1, wts)
    return out.reshape(2, _B, 1)


# speedup vs baseline: 1.0660x; 1.0206x over previous
import functools

import jax
import jax.numpy as jnp
from jax import lax
from jax.experimental import pallas as pl
from jax.experimental.pallas import tpu as pltpu
from jax.experimental.pallas import tpu_sc as plsc

_BETA = 0.5
_N = 4096
_B = 2048
_NE = 2 * _B
_NC = 2
_NS = 16
_NW = _NC * _NS
_EPW = _NE // _NW
_GB = 2
_NB = _EPW // _GB
_LANES = 16
_CH = _N // _LANES
_UNROLL = 4


def _sc_partials(A0v, A1v, idx, w0, w1):
    mesh = plsc.VectorSubcoreMesh(core_axis_name="c", subcore_axis_name="s")

    @functools.partial(
        pl.kernel,
        out_type=jax.ShapeDtypeStruct((_NE, _LANES), jnp.float32),
        mesh=mesh,
        scratch_types=[
            pltpu.VMEM((_GB * 2, _N), jnp.float32),
            pltpu.VMEM((_GB * 2, _N), jnp.float32),
            pltpu.VMEM((_GB * 2, _N), jnp.float32),
            pltpu.VMEM((_GB * 2, _N), jnp.float32),
            pltpu.VMEM((_N,), jnp.float32),
            pltpu.VMEM((_N,), jnp.float32),
            pltpu.VMEM((_NB, 2 * _GB), jnp.int32),
            pltpu.VMEM((_EPW, _LANES), jnp.float32),
            pltpu.SemaphoreType.DMA,
            pltpu.SemaphoreType.DMA,
        ],
    )
    def body(A0h, A1h, idxh, w0h, w1h, out_h,
             a0b0, a0b1, a1b0, a1b1, wv0, wv1, idx_v, res_v, sem0, sem1):
        wid = lax.axis_index("s") * _NC + lax.axis_index("c")
        bufs = ((a0b0, a1b0, sem0), (a0b1, a1b1, sem1))

        pltpu.sync_copy(idxh.at[wid], idx_v)
        pltpu.sync_copy(w0h, wv0)
        pltpu.sync_copy(w1h, wv1)

        def gathers(p, b):
            b0, b1, sem = bufs[p]
            return (pltpu.make_async_copy(A0h.at[idx_v.at[b]], b0, sem),
                    pltpu.make_async_copy(A1h.at[idx_v.at[b]], b1, sem))

        def start(p, b):
            g0, g1 = gathers(p, b)
            g0.start()
            g1.start()

        def wait(p, b):
            g0, g1 = gathers(p, b)
            g0.wait()
            g1.wait()

        start(0, 0)
        start(1, 1)

        zeros = jnp.zeros((_LANES,), jnp.float32)

        def edge_accs(b0, b1, i):
            def chunk(k, accs):
                f, r, s0s, s1s, s0d, s1d = accs
                for u in range(_UNROLL):
                    csl = pl.ds((k * _UNROLL + u) * _LANES, _LANES)
                    w0c = wv0[csl]
                    w1c = wv1[csl]
                    b0s = b0[2 * i, csl] * w0c
                    b1s = b1[2 * i, csl] * w1c
                    b0d = b0[2 * i + 1, csl] * w0c
                    b1d = b1[2 * i + 1, csl] * w1c
                    f = f + b0s * b1d
                    r = r + b1s * b0d
                    s0s = s0s + b0s * b0s
                    s1s = s1s + b1s * b1s
                    s0d = s0d + b0d * b0d
                    s1d = s1d + b1d * b1d
                return (f, r, s0s, s1s, s0d, s1d)
            return lax.fori_loop(0, _CH // _UNROLL, chunk, (zeros,) * 6)

        def run_batch(p, b):
            wait(p, b)
            b0, b1, _ = bufs[p][:3]
            lanes = lax.iota(jnp.int32, _LANES)

            def allsum(x):
                for s in (8, 4, 2, 1):
                    x = x + x.at[jnp.bitwise_xor(lanes, s)].get(
                        mode="promise_in_bounds")
                return x

            for i in range(_GB):
                accs = edge_accs(b0, b1, i)
                e_loc = b * _GB + i
                vec = zeros
                for q in range(6):
                    vec = jnp.where(lanes == q, allsum(accs[q]), vec)
                res_v[e_loc, :] = vec
            nb = b + 2

            @pl.when(nb < _NB)
            def _():
                start(p, nb)

        def g_body(g, carry):
            run_batch(0, 2 * g)
            run_batch(1, 2 * g + 1)
            return carry

        lax.fori_loop(0, _NB // 2, g_body, 0)
        pltpu.sync_copy(res_v, out_h.at[pl.ds(wid * _EPW, _EPW)])

    return body(A0v, A1v, idx, w0, w1)


def _finisher(P, mw):
    def body(p_ref, w10_ref, b10_ref, w20_ref, b20_ref,
             v10_ref, c10_ref, v20_ref, c20_ref,
             w11_ref, b11_ref, w21_ref, b21_ref,
             v11_ref, c11_ref, v21_ref, c21_ref, o_ref):
        p = p_ref[...]
        f, r, s0s, s1s, s0d, s1d = [p[:, q:q + 1] for q in range(6)]
        df = jnp.sqrt(s0s) * jnp.sqrt(s1d) + 1e-30
        dr = jnp.sqrt(s1s) * jnp.sqrt(s0d) + 1e-30
        on = f / df + r / dr

        def bf(x):
            return x.astype(jnp.bfloat16).astype(jnp.float32)

        def mlp(x, w1, b1, w2, b2, keepdims):
            h = jnp.maximum(x * w1[...] + b1[...], 0.0)
            s = jnp.sum(bf(h) * bf(w2[...]), axis=1, keepdims=keepdims)
            return s + (b2[...] if keepdims else b2[0, 0])

        on0 = on[:_B]
        on1 = on[_B:]
        sl0 = _BETA * mlp(on0, w10_ref, b10_ref, w20_ref, b20_ref, True)
        o_ref[pl.ds(0, _B)] = mlp(sl0, v10_ref, c10_ref, v20_ref, c20_ref,
                                  False)
        sl1 = _BETA * mlp(on1, w11_ref, b11_ref, w21_ref, b21_ref, True)
        o_ref[pl.ds(_B, _B)] = mlp(sl1, v11_ref, c11_ref, v21_ref, c21_ref,
                                   False)

    out = pl.pallas_call(
        body,
        out_shape=jax.ShapeDtypeStruct((_NE,), jnp.float32),
    )(P, *mw)
    return out


def kernel(A0, A1, feats0, feats1, out0, out1, src0, dst0, src1, dst1,
           p1w1_0, p1b1_0, p1w2_0, p1b2_0, p2w1_0, p2b1_0, p2w2_0, p2b2_0,
           p1w1_1, p1b1_1, p1w2_1, p1b2_1, p2w1_1, p2b1_1, p2w2_1, p2b2_1):
    src = jnp.concatenate([src0, src1])
    dst = jnp.concatenate([dst0, dst1])
    idx = jnp.stack([src, dst], axis=1)
    idx = idx.reshape(_NW, _NB, 2 * _GB).astype(jnp.int32)
    w0 = feats0.reshape(_N)
    w1 = feats1.reshape(_N)

    P = _sc_partials(A0, A1, idx, w0, w1)

    mw = (p1w1_0, p1b1_0.reshape(1, 64), p1w2_0.reshape(1, 64),
          p1b2_0.reshape(1, 1),
          p2w1_0, p2b1_0.reshape(1, 64), p2w2_0.reshape(1, 64),
          p2b2_0.reshape(1, 1),
          p1w1_1, p1b1_1.reshape(1, 64), p1w2_1.reshape(1, 64),
          p1b2_1.reshape(1, 1),
          p2w1_1, p2b1_1.reshape(1, 64), p2w2_1.reshape(1, 64),
          p2b2_1.reshape(1, 1))
    out = _finisher(P, mw)
    return out.reshape(2, _B, 1)
